# baseline (speedup 1.0000x reference)
import jax
import jax.numpy as jnp
from jax import lax
from jax.experimental import pallas as pl
from jax.experimental.pallas import tpu as pltpu
from jax.experimental.pallas import tpu_sc as plsc

NC = 2
NS = 16
NW = NC * NS
C = 112
CD = 128
GD = 16


def _round_up(x, m):
    return (x + m - 1) // m * m


def _deg_body(didx_hbm, out_hbm, slab_v, ones_v, zb_v, hist_s, sem, *, k2, np_):
    c = lax.axis_index("c")
    s = lax.axis_index("s")
    wid = s * NC + c
    for j in range(CD // 16):
        ones_v[pl.ds(16 * j, 16)] = jnp.full((16,), 1.0, jnp.float32)
    zpt = (2 * np_) // NS

    def zrow(i, carry):
        zb_v[pl.ds(16 * i, 16)] = jnp.zeros((16,), jnp.float32)
        return carry

    lax.fori_loop(0, zpt // 16, zrow, 0)
    pltpu.sync_copy(zb_v, hist_s.at[pl.ds(s * zpt, zpt)])
    pltpu.sync_copy(didx_hbm.at[wid], slab_v)
    plsc.subcore_barrier()

    def grp(g, carry):
        hs = [
            pltpu.async_copy(ones_v, hist_s.at[slab_v.at[g * GD + j]], sem, add=True)
            for j in range(GD)
        ]
        for h in hs:
            h.wait()
        return carry

    lax.fori_loop(0, k2 // GD, grp, 0)
    plsc.subcore_barrier()
    pltpu.sync_copy(hist_s.at[pl.ds(s * zpt, zpt)], zb_v)
    pltpu.sync_copy(zb_v, out_hbm.at[pl.ds(c * 2 * np_ + s * zpt, zpt)])


def _msg_body(ht_hbm, src_hbm, dst_hbm, out_hbm,
              sidx_v, didx_v, msg_v, acc_s, isem, gsem, ssem, *, k, np_, d):
    c = lax.axis_index("c")
    s = lax.axis_index("s")
    wid = s * NC + c
    rpt = np_ // NS

    def zrow(i, carry):
        for j in range(d // 16):
            msg_v[0, i, pl.ds(16 * j, 16)] = jnp.zeros((16,), jnp.float32)
        return carry

    lax.fori_loop(0, C, zrow, 0)
    zh = []
    off = 0
    while off < rpt:
        n = min(C, rpt - off)
        zh.append(pltpu.async_copy(msg_v.at[0, pl.ds(0, n)],
                                   acc_s.at[pl.ds(s * rpt + off, n)], gsem))
        off += n
    for h in zh:
        h.wait()
    plsc.subcore_barrier()

    def _fire_idx(t, slot):
        pltpu.async_copy(src_hbm.at[wid, t], sidx_v.at[slot], isem)
        pltpu.async_copy(dst_hbm.at[wid, t], didx_v.at[slot], isem)

    def _wait_idx(slot):
        pltpu.make_async_copy(src_hbm.at[wid, 0], sidx_v.at[slot], isem).wait()
        pltpu.make_async_copy(dst_hbm.at[wid, 0], didx_v.at[slot], isem).wait()

    def _fire_gather(slot, mb):
        pltpu.async_copy(ht_hbm.at[sidx_v.at[slot]], msg_v.at[mb], gsem)

    def _wait_gather(mb):
        pltpu.make_async_copy(ht_hbm.at[sidx_v.at[0]], msg_v.at[mb], gsem).wait()

    def _fire_scat(slot, mb):
        pltpu.async_copy(msg_v.at[mb], acc_s.at[didx_v.at[slot]], ssem, add=True)

    def _wait_scat(mb):
        pltpu.make_async_copy(msg_v.at[mb], acc_s.at[didx_v.at[0]], ssem).wait()

    for t in range(3):
        _fire_idx(t, t)

    def body(g, carry):
        for bb in range(6):
            t = 6 * g + bb
            mb = bb % 3

            @pl.when(t >= 3)
            def _(mb=mb):
                _wait_scat(mb)

            _wait_idx(bb)
            _fire_gather(bb, mb)

            @pl.when(t >= 1)
            def _(pslot=(bb + 5) % 6, pmb=(bb + 2) % 3):
                _wait_gather(pmb)
                _fire_scat(pslot, pmb)

            @pl.when(t + 3 <= k - 1)
            def _(t=t, slot=(bb + 3) % 6):
                _fire_idx(t + 3, slot)
        return carry

    lax.fori_loop(0, k // 6, body, 0)
    _wait_gather((k - 1) % 3)
    _fire_scat((k - 1) % 6, (k - 1) % 3)
    _wait_scat((k - 3) % 3)
    _wait_scat((k - 2) % 3)
    _wait_scat((k - 1) % 3)
    plsc.subcore_barrier()
    chunks = []
    off = 0
    while off < rpt:
        chunks.append((off, min(C, rpt - off)))
        off += chunks[-1][1]
    rh = []
    wh = []
    for i, (o, n) in enumerate(chunks):
        if i >= 3:
            wh[i - 3].wait()
        rh.append(pltpu.async_copy(acc_s.at[pl.ds(s * rpt + o, n)],
                                   msg_v.at[i % 3, pl.ds(0, n)], gsem))
        if i >= 2:
            po, pn = chunks[i - 2]
            rh[i - 2].wait()
            wh.append(pltpu.async_copy(msg_v.at[(i - 2) % 3, pl.ds(0, pn)],
                                       out_hbm.at[c, pl.ds(s * rpt + po, pn)],
                                       ssem))
    for i in range(max(0, len(chunks) - 2), len(chunks)):
        o, n = chunks[i]
        rh[i].wait()
        wh.append(pltpu.async_copy(msg_v.at[i % 3, pl.ds(0, n)],
                                   out_hbm.at[c, pl.ds(s * rpt + o, n)], ssem))
    for i in range(max(0, len(chunks) - 3), len(chunks)):
        wh[i].wait()


def _norms(degT_ref):
    dgo = degT_ref[:, 0:1] + degT_ref[:, 2:3]
    dgi = degT_ref[:, 1:2] + degT_ref[:, 3:4]
    ns = jnp.where(dgo > 0, lax.rsqrt(jnp.maximum(dgo, 1e-12)), 0.0)
    nd = jnp.where(dgi > 0, lax.rsqrt(jnp.maximum(dgi, 1e-12)), 0.0)
    return ns, nd


def _mm1_body(degT_ref, x_ref, w_ref, o_ref):
    ns, _ = _norms(degT_ref)
    o_ref[...] = lax.dot_general(x_ref[...] * ns, w_ref[...],
                                 (((1,), (0,)), ((), ())),
                                 preferred_element_type=jnp.float32)


def _mid_body(degT_ref, p_ref, b_ref, a_ref, w_ref, oh_ref, ot_ref):
    ns, nd = _norms(degT_ref)
    h = (p_ref[0] + p_ref[1]) * nd + b_ref[...]
    h = jnp.where(h > 0, h, h * a_ref[...])
    oh_ref[...] = h
    ot_ref[...] = lax.dot_general(h * ns, w_ref[...], (((1,), (0,)), ((), ())),
                                  preferred_element_type=jnp.float32)


def kernel(features, edge_index, W1, b1, W2, b2, a):
    N, D = features.shape
    E = edge_index.shape[1]
    Np = _round_up(N, 128)
    if Np - N < 16:
        Np += 128
    n_dummy = Np - N

    K = _round_up(-(-E // (NW * C)), 6)
    K2 = _round_up(-(-(2 * E) // (NW * CD)), GD)

    f32 = jnp.float32
    src = edge_index[0]
    dst = edge_index[1]

    pad_e = NW * K * C - E
    pad_ids = N + (jnp.arange(pad_e, dtype=jnp.int32) % n_dummy)
    srcp = jnp.concatenate([src, pad_ids]).reshape(NW, K, C)
    dstp = jnp.concatenate([dst, pad_ids]).reshape(NW, K, C)

    pad2 = NW * K2 * CD - 2 * E
    pad2_ids = N + (jnp.arange(pad2, dtype=jnp.int32) % n_dummy)
    didx = jnp.concatenate([src, dst + Np, pad2_ids]).reshape(NW, K2, CD)

    mesh = plsc.VectorSubcoreMesh(core_axis_name="c", subcore_axis_name="s",
                                  num_cores=NC, num_subcores=NS)

    deg_call = pl.kernel(
        lambda *refs: _deg_body(*refs, k2=K2, np_=Np),
        out_type=jax.ShapeDtypeStruct((NC * 2 * Np,), f32),
        mesh=mesh,
        scratch_types=[
            pltpu.VMEM((K2, CD), jnp.int32),
            pltpu.VMEM((CD,), f32),
            pltpu.VMEM(((2 * Np) // NS,), f32),
            pltpu.VMEM_SHARED((2 * Np,), f32),
            pltpu.SemaphoreType.DMA,
        ],
    )
    deg2 = deg_call(didx)
    degT = deg2.reshape(NC, 2, Np).transpose(2, 0, 1).reshape(Np, NC * 2)

    msg_call = pl.kernel(
        lambda *refs: _msg_body(*refs, k=K, np_=Np, d=D),
        out_type=jax.ShapeDtypeStruct((NC, Np, D), f32),
        mesh=mesh,
        scratch_types=[
            pltpu.VMEM((6, C), jnp.int32),
            pltpu.VMEM((6, C), jnp.int32),
            pltpu.VMEM((3, C, D), f32),
            pltpu.VMEM_SHARED((Np, D), f32),
            pltpu.SemaphoreType.DMA,
            pltpu.SemaphoreType.DMA,
            pltpu.SemaphoreType.DMA,
        ],
    )

    BM = 1264
    grid = (Np // BM,)
    degT_spec = pl.BlockSpec((BM, NC * 2), lambda m: (m, 0))
    row_spec = pl.BlockSpec((BM, D), lambda m: (m, 0))
    w_spec = pl.BlockSpec((D, D), lambda m: (0, 0))
    vec_spec = pl.BlockSpec((1, D), lambda m: (0, 0))
    p_spec = pl.BlockSpec((NC, BM, D), lambda m: (0, m, 0))

    b1r = b1.reshape(1, D)
    b2r = b2.reshape(1, D)
    ar = a.reshape(1, D)

    mm1_call = pl.pallas_call(
        _mm1_body, grid=grid,
        in_specs=[degT_spec, row_spec, w_spec],
        out_specs=row_spec,
        out_shape=jax.ShapeDtypeStruct((Np, D), f32),
    )
    mid_call = pl.pallas_call(
        _mid_body, grid=grid,
        in_specs=[degT_spec, p_spec, vec_spec, vec_spec, w_spec],
        out_specs=[row_spec, row_spec],
        out_shape=[jax.ShapeDtypeStruct((N, D), f32),
                   jax.ShapeDtypeStruct((Np, D), f32)],
    )

    ht1 = mm1_call(degT, features, W1)
    p1 = msg_call(ht1, srcp, dstp)
    _, ht2 = mid_call(degT, p1, b1r, ar, W2)
    p2 = msg_call(ht2, srcp, dstp)
    h2, _ = mid_call(degT, p2, b2r, ar, W2)
    return h2


# speedup vs baseline: 1.0154x; 1.0154x over previous
import jax
import jax.numpy as jnp
from jax import lax
from jax.experimental import pallas as pl
from jax.experimental.pallas import tpu as pltpu
from jax.experimental.pallas import tpu_sc as plsc

NC = 2
NS = 16
NW = NC * NS
C = 112
CD = 128
GD = 16


def _round_up(x, m):
    return (x + m - 1) // m * m


def _deg_body(didx_hbm, out_hbm, slab_v, ones_v, zb_v, hist_s, sem, *, k2, np_):
    c = lax.axis_index("c")
    s = lax.axis_index("s")
    wid = s * NC + c
    for j in range(CD // 16):
        ones_v[pl.ds(16 * j, 16)] = jnp.full((16,), 1.0, jnp.float32)
    zpt = (2 * np_) // NS

    def zrow(i, carry):
        zb_v[pl.ds(16 * i, 16)] = jnp.zeros((16,), jnp.float32)
        return carry

    lax.fori_loop(0, zpt // 16, zrow, 0)
    pltpu.sync_copy(zb_v, hist_s.at[pl.ds(s * zpt, zpt)])
    pltpu.sync_copy(didx_hbm.at[wid], slab_v)
    plsc.subcore_barrier()

    def grp(g, carry):
        hs = [
            pltpu.async_copy(ones_v, hist_s.at[slab_v.at[g * GD + j]], sem, add=True)
            for j in range(GD)
        ]
        for h in hs:
            h.wait()
        return carry

    lax.fori_loop(0, k2 // GD, grp, 0)
    plsc.subcore_barrier()
    pltpu.sync_copy(hist_s.at[pl.ds(s * zpt, zpt)], zb_v)
    pltpu.sync_copy(zb_v, out_hbm.at[pl.ds(c * 2 * np_ + s * zpt, zpt)])


def _msg_body(ht_hbm, src_hbm, dst_hbm, out_hbm,
              sidx_v, didx_v, msg_v, acc_s, isem, gsem, ssem, *, k, np_, d):
    c = lax.axis_index("c")
    s = lax.axis_index("s")
    wid = s * NC + c
    rpt = np_ // NS

    def zrow(i, carry):
        for j in range(d // 16):
            msg_v[0, i, pl.ds(16 * j, 16)] = jnp.zeros((16,), jnp.float32)
        return carry

    lax.fori_loop(0, C, zrow, 0)
    zh = []
    off = 0
    while off < rpt:
        n = min(C, rpt - off)
        zh.append(pltpu.async_copy(msg_v.at[0, pl.ds(0, n)],
                                   acc_s.at[pl.ds(s * rpt + off, n)], gsem))
        off += n
    for h in zh:
        h.wait()
    plsc.subcore_barrier()

    def _fire_idx(t, slot):
        pltpu.async_copy(src_hbm.at[wid, t], sidx_v.at[slot], isem)
        pltpu.async_copy(dst_hbm.at[wid, t], didx_v.at[slot], isem)

    def _wait_idx(slot):
        pltpu.make_async_copy(src_hbm.at[wid, 0], sidx_v.at[slot], isem).wait()
        pltpu.make_async_copy(dst_hbm.at[wid, 0], didx_v.at[slot], isem).wait()

    def _fire_gather(slot, mb):
        pltpu.async_copy(ht_hbm.at[sidx_v.at[slot]], msg_v.at[mb], gsem)

    def _wait_gather(mb):
        pltpu.make_async_copy(ht_hbm.at[sidx_v.at[0]], msg_v.at[mb], gsem).wait()

    def _fire_scat(slot, mb):
        pltpu.async_copy(msg_v.at[mb], acc_s.at[didx_v.at[slot]], ssem, add=True)

    def _wait_scat(mb):
        pltpu.make_async_copy(msg_v.at[mb], acc_s.at[didx_v.at[0]], ssem).wait()

    for t in range(3):
        _fire_idx(t, t)

    def body(g, carry):
        for bb in range(6):
            t = 6 * g + bb
            mb = bb % 3

            @pl.when(t >= 3)
            def _(mb=mb):
                _wait_scat(mb)

            _wait_idx(bb)
            _fire_gather(bb, mb)

            @pl.when(t >= 1)
            def _(pslot=(bb + 5) % 6, pmb=(bb + 2) % 3):
                _wait_gather(pmb)
                _fire_scat(pslot, pmb)

            @pl.when(t + 3 <= k - 1)
            def _(t=t, slot=(bb + 3) % 6):
                _fire_idx(t + 3, slot)
        return carry

    lax.fori_loop(0, k // 6, body, 0)
    _wait_gather((k - 1) % 3)
    _fire_scat((k - 1) % 6, (k - 1) % 3)
    _wait_scat((k - 3) % 3)
    _wait_scat((k - 2) % 3)
    _wait_scat((k - 1) % 3)
    plsc.subcore_barrier()
    chunks = []
    off = 0
    while off < rpt:
        chunks.append((off, min(C, rpt - off)))
        off += chunks[-1][1]
    rh = []
    wh = []
    for i, (o, n) in enumerate(chunks):
        if i >= 3:
            wh[i - 3].wait()
        rh.append(pltpu.async_copy(acc_s.at[pl.ds(s * rpt + o, n)],
                                   msg_v.at[i % 3, pl.ds(0, n)], gsem))
        if i >= 2:
            po, pn = chunks[i - 2]
            rh[i - 2].wait()
            wh.append(pltpu.async_copy(msg_v.at[(i - 2) % 3, pl.ds(0, pn)],
                                       out_hbm.at[c, pl.ds(s * rpt + po, pn)],
                                       ssem))
    for i in range(max(0, len(chunks) - 2), len(chunks)):
        o, n = chunks[i]
        rh[i].wait()
        wh.append(pltpu.async_copy(msg_v.at[i % 3, pl.ds(0, n)],
                                   out_hbm.at[c, pl.ds(s * rpt + o, n)], ssem))
    for i in range(max(0, len(chunks) - 3), len(chunks)):
        wh[i].wait()


def _norms(degT_ref):
    dgo = degT_ref[:, 0:1] + degT_ref[:, 2:3]
    dgi = degT_ref[:, 1:2] + degT_ref[:, 3:4]
    ns = jnp.where(dgo > 0, lax.rsqrt(jnp.maximum(dgo, 1e-12)), 0.0)
    nd = jnp.where(dgi > 0, lax.rsqrt(jnp.maximum(dgi, 1e-12)), 0.0)
    return ns, nd


def _mm1_body(degT_ref, x_ref, w_ref, o_ref):
    ns, _ = _norms(degT_ref)
    o_ref[...] = lax.dot_general(x_ref[...] * ns, w_ref[...],
                                 (((1,), (0,)), ((), ())),
                                 preferred_element_type=jnp.float32)


def _mid_body(degT_ref, p_ref, b_ref, a_ref, w_ref, ot_ref):
    ns, nd = _norms(degT_ref)
    h = (p_ref[0] + p_ref[1]) * nd + b_ref[...]
    h = jnp.where(h > 0, h, h * a_ref[...])
    ot_ref[...] = lax.dot_general(h * ns, w_ref[...], (((1,), (0,)), ((), ())),
                                  preferred_element_type=jnp.float32)


def _fin_body(degT_ref, p_ref, b_ref, a_ref, oh_ref):
    _, nd = _norms(degT_ref)
    h = (p_ref[0] + p_ref[1]) * nd + b_ref[...]
    oh_ref[...] = jnp.where(h > 0, h, h * a_ref[...])


def kernel(features, edge_index, W1, b1, W2, b2, a):
    N, D = features.shape
    E = edge_index.shape[1]
    Np = _round_up(N, 128)
    if Np - N < 16:
        Np += 128
    n_dummy = Np - N

    K = _round_up(-(-E // (NW * C)), 6)
    K2 = _round_up(-(-(2 * E) // (NW * CD)), GD)

    f32 = jnp.float32
    src = edge_index[0]
    dst = edge_index[1]

    pad_e = NW * K * C - E
    pad_ids = N + (jnp.arange(pad_e, dtype=jnp.int32) % n_dummy)
    srcp = jnp.concatenate([src, pad_ids]).reshape(NW, K, C)
    dstp = jnp.concatenate([dst, pad_ids]).reshape(NW, K, C)

    pad2 = NW * K2 * CD - 2 * E
    pad2_ids = N + (jnp.arange(pad2, dtype=jnp.int32) % n_dummy)
    didx = jnp.concatenate([src, dst + Np, pad2_ids]).reshape(NW, K2, CD)

    mesh = plsc.VectorSubcoreMesh(core_axis_name="c", subcore_axis_name="s",
                                  num_cores=NC, num_subcores=NS)

    deg_call = pl.kernel(
        lambda *refs: _deg_body(*refs, k2=K2, np_=Np),
        out_type=jax.ShapeDtypeStruct((NC * 2 * Np,), f32),
        mesh=mesh,
        scratch_types=[
            pltpu.VMEM((K2, CD), jnp.int32),
            pltpu.VMEM((CD,), f32),
            pltpu.VMEM(((2 * Np) // NS,), f32),
            pltpu.VMEM_SHARED((2 * Np,), f32),
            pltpu.SemaphoreType.DMA,
        ],
    )
    deg2 = deg_call(didx)
    degT = deg2.reshape(NC, 2, Np).transpose(2, 0, 1).reshape(Np, NC * 2)

    msg_call = pl.kernel(
        lambda *refs: _msg_body(*refs, k=K, np_=Np, d=D),
        out_type=jax.ShapeDtypeStruct((NC, Np, D), f32),
        mesh=mesh,
        scratch_types=[
            pltpu.VMEM((6, C), jnp.int32),
            pltpu.VMEM((6, C), jnp.int32),
            pltpu.VMEM((3, C, D), f32),
            pltpu.VMEM_SHARED((Np, D), f32),
            pltpu.SemaphoreType.DMA,
            pltpu.SemaphoreType.DMA,
            pltpu.SemaphoreType.DMA,
        ],
    )

    BM = 1264
    grid = (Np // BM,)
    degT_spec = pl.BlockSpec((BM, NC * 2), lambda m: (m, 0))
    row_spec = pl.BlockSpec((BM, D), lambda m: (m, 0))
    w_spec = pl.BlockSpec((D, D), lambda m: (0, 0))
    vec_spec = pl.BlockSpec((1, D), lambda m: (0, 0))
    p_spec = pl.BlockSpec((NC, BM, D), lambda m: (0, m, 0))

    b1r = b1.reshape(1, D)
    b2r = b2.reshape(1, D)
    ar = a.reshape(1, D)

    mm1_call = pl.pallas_call(
        _mm1_body, grid=grid,
        in_specs=[degT_spec, row_spec, w_spec],
        out_specs=row_spec,
        out_shape=jax.ShapeDtypeStruct((Np, D), f32),
    )
    mid_call = pl.pallas_call(
        _mid_body, grid=grid,
        in_specs=[degT_spec, p_spec, vec_spec, vec_spec, w_spec],
        out_specs=row_spec,
        out_shape=jax.ShapeDtypeStruct((Np, D), f32),
    )
    fin_call = pl.pallas_call(
        _fin_body, grid=grid,
        in_specs=[degT_spec, p_spec, vec_spec, vec_spec],
        out_specs=row_spec,
        out_shape=jax.ShapeDtypeStruct((N, D), f32),
    )

    ht1 = mm1_call(degT, features, W1)
    p1 = msg_call(ht1, srcp, dstp)
    ht2 = mid_call(degT, p1, b1r, ar, W2)
    p2 = msg_call(ht2, srcp, dstp)
    return fin_call(degT, p2, b2r, ar)
